# baseline (speedup 1.0000x reference)
import functools

import jax
import jax.numpy as jnp
from jax import lax
from jax.experimental import pallas as pl
from jax.experimental.pallas import tpu as pltpu
from jax.experimental.pallas import tpu_sc as plsc

N_NODES = 10000
D_FEAT = 128
N_EDGES = 320000

NC = 2
NS = 16
NW = NC * NS
CHUNK = 128

ROWS_PER_TILE = 624
REM_ROW0 = NS * ROWS_PER_TILE
REM_ROWS = N_NODES - REM_ROW0
CHUNKS_PER_TILE = 80
EDGES_PER_TILE = CHUNKS_PER_TILE * CHUNK
E_PAD = NW * EDGES_PER_TILE
ACC_ROWS = N_NODES + 8
NBUF = 2
PHASES = 2
CH_PER_PHASE = CHUNKS_PER_TILE // PHASES


def _sc_body(x_hbm, src_hbm, dst_hbm, zeros_hbm, out_hbm,
             acc, src_all, dst_all, rows, *gsems):
    cid = lax.axis_index("c")
    sid = lax.axis_index("s")
    row0 = sid * ROWS_PER_TILE

    @pl.when(cid == 0)
    def _():
        pltpu.sync_copy(x_hbm.at[pl.ds(row0, ROWS_PER_TILE)],
                        acc.at[pl.ds(row0, ROWS_PER_TILE)])

    @pl.when(cid == 1)
    def _():
        pltpu.sync_copy(zeros_hbm, acc.at[pl.ds(row0, ROWS_PER_TILE)])

    @pl.when((cid == 0) & (sid == NS - 1))
    def _():
        pltpu.sync_copy(x_hbm.at[pl.ds(REM_ROW0, REM_ROWS)],
                        acc.at[pl.ds(REM_ROW0, REM_ROWS)])

    @pl.when((cid == 1) & (sid == NS - 1))
    def _():
        pltpu.sync_copy(zeros_hbm.at[pl.ds(0, REM_ROWS)],
                        acc.at[pl.ds(REM_ROW0, REM_ROWS)])

    plsc.subcore_barrier()

    wid = sid * NC + cid

    groups = CH_PER_PHASE // NBUF
    for phase in range(PHASES):
        base = wid * CHUNKS_PER_TILE + phase * CH_PER_PHASE
        pltpu.sync_copy(src_hbm.at[pl.ds(base, CH_PER_PHASE)], src_all)
        pltpu.sync_copy(dst_hbm.at[pl.ds(base, CH_PER_PHASE)], dst_all)

        for k in range(NBUF):
            pltpu.async_copy(x_hbm.at[src_all.at[k]], rows.at[k], gsems[k])

        def group(g, carry):
            for k in range(NBUF):
                j = g * NBUF + k
                pltpu.make_async_copy(x_hbm.at[src_all.at[j]], rows.at[k],
                                      gsems[k]).wait()
                pltpu.sync_copy(rows.at[k], acc.at[dst_all.at[j]], add=True)

                @pl.when(g < groups - 1)
                def _():
                    pltpu.async_copy(x_hbm.at[src_all.at[j + NBUF]],
                                     rows.at[k], gsems[k])
            return carry

        lax.fori_loop(0, groups, group, 0)

    plsc.subcore_barrier()

    pltpu.sync_copy(acc.at[pl.ds(row0, ROWS_PER_TILE)],
                    out_hbm.at[pl.ds(cid * N_NODES + row0, ROWS_PER_TILE)])

    @pl.when(sid == NS - 1)
    def _():
        pltpu.sync_copy(acc.at[pl.ds(REM_ROW0, REM_ROWS)],
                        out_hbm.at[pl.ds(cid * N_NODES + REM_ROW0, REM_ROWS)])


@jax.jit
def _sc_aggregate(x, src, dst, zeros):
    mesh = plsc.VectorSubcoreMesh(core_axis_name="c", subcore_axis_name="s")
    k = pl.kernel(
        _sc_body,
        out_type=jax.ShapeDtypeStruct((NC * N_NODES, D_FEAT), jnp.float32),
        mesh=mesh,
        scratch_types=[
            pltpu.VMEM_SHARED((ACC_ROWS, D_FEAT), jnp.float32),
            pltpu.VMEM((CH_PER_PHASE, CHUNK), jnp.int32),
            pltpu.VMEM((CH_PER_PHASE, CHUNK), jnp.int32),
            pltpu.VMEM((NBUF, CHUNK, D_FEAT), jnp.float32),
        ] + [pltpu.SemaphoreType.DMA] * NBUF,
    )
    return k(x, src, dst, zeros)


def _mm_body(p0_ref, p1_ref, w_ref, b_ref, out_ref):
    h = p0_ref[...] + p1_ref[...]
    out = jnp.dot(h, w_ref[...], preferred_element_type=jnp.float32)
    out_ref[...] = jnp.maximum(out + b_ref[...], 0.0)


BLOCK_M = 1000


@jax.jit
def _mm(p0, p1, W, b2d):
    grid = (N_NODES // BLOCK_M,)
    return pl.pallas_call(
        _mm_body,
        grid=grid,
        in_specs=[
            pl.BlockSpec((BLOCK_M, D_FEAT), lambda i: (i, 0)),
            pl.BlockSpec((BLOCK_M, D_FEAT), lambda i: (i, 0)),
            pl.BlockSpec((D_FEAT, D_FEAT), lambda i: (0, 0)),
            pl.BlockSpec((1, D_FEAT), lambda i: (0, 0)),
        ],
        out_specs=pl.BlockSpec((BLOCK_M, D_FEAT), lambda i: (i, 0)),
        out_shape=jax.ShapeDtypeStruct((N_NODES, D_FEAT), jnp.float32),
    )(p0, p1, W, b2d)


def kernel(inputs, edge_index, W, b):
    src = edge_index[0].astype(jnp.int32)
    dst = edge_index[1].astype(jnp.int32)
    pad = E_PAD - N_EDGES
    src = jnp.concatenate([src, jnp.zeros((pad,), jnp.int32)])
    dst = jnp.concatenate([dst, jnp.full((pad,), N_NODES, jnp.int32)])
    src = src.reshape(NW * CHUNKS_PER_TILE, CHUNK)
    dst = dst.reshape(NW * CHUNKS_PER_TILE, CHUNK)
    zeros = jnp.zeros((ROWS_PER_TILE, D_FEAT), jnp.float32)
    parts = _sc_aggregate(inputs, src, dst, zeros)
    return _mm(parts[:N_NODES], parts[N_NODES:], W, b.reshape(1, D_FEAT))


# speedup vs baseline: 1.2978x; 1.2978x over previous
import functools

import jax
import jax.numpy as jnp
from jax import lax
from jax.experimental import pallas as pl
from jax.experimental.pallas import tpu as pltpu
from jax.experimental.pallas import tpu_sc as plsc

N_NODES = 10000
D_FEAT = 128
N_EDGES = 320000

NC = 2
NS = 16
NW = NC * NS
CHUNK = 128

ROWS_PER_TILE = 624
REM_ROW0 = NS * ROWS_PER_TILE
REM_ROWS = N_NODES - REM_ROW0
CHUNKS_PER_TILE = 80
EDGES_PER_TILE = CHUNKS_PER_TILE * CHUNK
E_PAD = NW * EDGES_PER_TILE
ACC_ROWS = N_NODES + 16
NBUF = 2
PHASES = 2
CH_PER_PHASE = CHUNKS_PER_TILE // PHASES


def _sc_body(x_hbm, src_hbm, dst_hbm, zeros_hbm, out_hbm,
             acc, src_all, dst_all, rows, *gsems):
    cid = lax.axis_index("c")
    sid = lax.axis_index("s")
    row0 = sid * ROWS_PER_TILE

    @pl.when(cid == 0)
    def _():
        pltpu.sync_copy(x_hbm.at[pl.ds(row0, ROWS_PER_TILE)],
                        acc.at[pl.ds(row0, ROWS_PER_TILE)])

    @pl.when(cid == 1)
    def _():
        pltpu.sync_copy(zeros_hbm, acc.at[pl.ds(row0, ROWS_PER_TILE)])

    @pl.when((cid == 0) & (sid == NS - 1))
    def _():
        pltpu.sync_copy(x_hbm.at[pl.ds(REM_ROW0, REM_ROWS)],
                        acc.at[pl.ds(REM_ROW0, REM_ROWS)])

    @pl.when((cid == 1) & (sid == NS - 1))
    def _():
        pltpu.sync_copy(zeros_hbm.at[pl.ds(0, REM_ROWS)],
                        acc.at[pl.ds(REM_ROW0, REM_ROWS)])

    plsc.subcore_barrier()

    wid = sid * NC + cid

    groups = CH_PER_PHASE // NBUF
    for phase in range(PHASES):
        base = wid * CHUNKS_PER_TILE + phase * CH_PER_PHASE
        pltpu.sync_copy(src_hbm.at[pl.ds(base, CH_PER_PHASE)], src_all)
        pltpu.sync_copy(dst_hbm.at[pl.ds(base, CH_PER_PHASE)], dst_all)

        for k in range(NBUF):
            pltpu.async_copy(x_hbm.at[src_all.at[k]], rows.at[k], gsems[k])

        def group(g, carry):
            for k in range(NBUF):
                j = g * NBUF + k
                pltpu.make_async_copy(x_hbm.at[src_all.at[j]], rows.at[k],
                                      gsems[k]).wait()
                pltpu.sync_copy(rows.at[k], acc.at[dst_all.at[j]], add=True)

                @pl.when(g < groups - 1)
                def _():
                    pltpu.async_copy(x_hbm.at[src_all.at[j + NBUF]],
                                     rows.at[k], gsems[k])
            return carry

        lax.fori_loop(0, groups, group, 0)

    plsc.subcore_barrier()

    pltpu.sync_copy(acc.at[pl.ds(row0, ROWS_PER_TILE)],
                    out_hbm.at[pl.ds(cid * N_NODES + row0, ROWS_PER_TILE)])

    @pl.when(sid == NS - 1)
    def _():
        pltpu.sync_copy(acc.at[pl.ds(REM_ROW0, REM_ROWS)],
                        out_hbm.at[pl.ds(cid * N_NODES + REM_ROW0, REM_ROWS)])


@jax.jit
def _sc_aggregate(x, src, dst, zeros):
    mesh = plsc.VectorSubcoreMesh(core_axis_name="c", subcore_axis_name="s")
    k = pl.kernel(
        _sc_body,
        out_type=jax.ShapeDtypeStruct((NC * N_NODES, D_FEAT), jnp.float32),
        mesh=mesh,
        scratch_types=[
            pltpu.VMEM_SHARED((ACC_ROWS, D_FEAT), jnp.float32),
            pltpu.VMEM((CH_PER_PHASE, CHUNK), jnp.int32),
            pltpu.VMEM((CH_PER_PHASE, CHUNK), jnp.int32),
            pltpu.VMEM((NBUF, CHUNK, D_FEAT), jnp.float32),
        ] + [pltpu.SemaphoreType.DMA] * NBUF,
    )
    return k(x, src, dst, zeros)


def _mm_body(p0_ref, p1_ref, w_ref, b_ref, out_ref):
    h = p0_ref[...] + p1_ref[...]
    out = jnp.dot(h, w_ref[...], preferred_element_type=jnp.float32)
    out_ref[...] = jnp.maximum(out + b_ref[...], 0.0)


BLOCK_M = 1000


@jax.jit
def _mm(p0, p1, W, b2d):
    grid = (N_NODES // BLOCK_M,)
    return pl.pallas_call(
        _mm_body,
        grid=grid,
        in_specs=[
            pl.BlockSpec((BLOCK_M, D_FEAT), lambda i: (i, 0)),
            pl.BlockSpec((BLOCK_M, D_FEAT), lambda i: (i, 0)),
            pl.BlockSpec((D_FEAT, D_FEAT), lambda i: (0, 0)),
            pl.BlockSpec((1, D_FEAT), lambda i: (0, 0)),
        ],
        out_specs=pl.BlockSpec((BLOCK_M, D_FEAT), lambda i: (i, 0)),
        out_shape=jax.ShapeDtypeStruct((N_NODES, D_FEAT), jnp.float32),
    )(p0, p1, W, b2d)


def kernel(inputs, edge_index, W, b):
    src = edge_index[0].astype(jnp.int32)
    dst = edge_index[1].astype(jnp.int32)
    per_tile = N_EDGES // NW
    pad_per_tile = EDGES_PER_TILE - per_tile
    pad_dst = jnp.broadcast_to(
        N_NODES + (jnp.arange(pad_per_tile, dtype=jnp.int32) % 16),
        (NW, pad_per_tile))
    src = jnp.concatenate(
        [src.reshape(NW, per_tile),
         jnp.zeros((NW, pad_per_tile), jnp.int32)], axis=1)
    dst = jnp.concatenate([dst.reshape(NW, per_tile), pad_dst], axis=1)
    src = src.reshape(NW * CHUNKS_PER_TILE, CHUNK)
    dst = dst.reshape(NW * CHUNKS_PER_TILE, CHUNK)
    zeros = jnp.zeros((ROWS_PER_TILE, D_FEAT), jnp.float32)
    parts = _sc_aggregate(inputs, src, dst, zeros)
    return _mm(parts[:N_NODES], parts[N_NODES:], W, b.reshape(1, D_FEAT))


# speedup vs baseline: 1.3105x; 1.0098x over previous
import functools

import jax
import jax.numpy as jnp
from jax import lax
from jax.experimental import pallas as pl
from jax.experimental.pallas import tpu as pltpu
from jax.experimental.pallas import tpu_sc as plsc

N_NODES = 10000
D_FEAT = 128
N_EDGES = 320000

NC = 2
NS = 16
NW = NC * NS
CHUNK = 128

ROWS_PER_TILE = 624
REM_ROW0 = NS * ROWS_PER_TILE
REM_ROWS = N_NODES - REM_ROW0
CHUNKS_PER_TILE = 80
EDGES_PER_TILE = CHUNKS_PER_TILE * CHUNK
E_PAD = NW * EDGES_PER_TILE
ACC_ROWS = N_NODES + 16
NBUF = 3
IDXBUF = 6


def _sc_body(x_hbm, idx_hbm, zeros_hbm, out_hbm, acc, idxb, rows, *sems):
    gsems = sems[0:NBUF]
    ssems = sems[NBUF:2 * NBUF]
    isems = sems[2 * NBUF:2 * NBUF + IDXBUF]
    cid = lax.axis_index("c")
    sid = lax.axis_index("s")
    row0 = sid * ROWS_PER_TILE

    @pl.when(cid == 0)
    def _():
        pltpu.sync_copy(x_hbm.at[pl.ds(row0, ROWS_PER_TILE)],
                        acc.at[pl.ds(row0, ROWS_PER_TILE)])

    @pl.when(cid == 1)
    def _():
        pltpu.sync_copy(zeros_hbm, acc.at[pl.ds(row0, ROWS_PER_TILE)])

    @pl.when((cid == 0) & (sid == NS - 1))
    def _():
        pltpu.sync_copy(x_hbm.at[pl.ds(REM_ROW0, REM_ROWS)],
                        acc.at[pl.ds(REM_ROW0, REM_ROWS)])

    @pl.when((cid == 1) & (sid == NS - 1))
    def _():
        pltpu.sync_copy(zeros_hbm.at[pl.ds(0, REM_ROWS)],
                        acc.at[pl.ds(REM_ROW0, REM_ROWS)])

    plsc.subcore_barrier()

    wid = sid * NC + cid
    base = wid * CHUNKS_PER_TILE
    CPT = CHUNKS_PER_TILE

    def idx_load(j, kk):
        pltpu.async_copy(idx_hbm.at[base + j], idxb.at[kk], isems[kk])

    def idx_wait(j, kk):
        pltpu.make_async_copy(idx_hbm.at[base + j], idxb.at[kk],
                              isems[kk]).wait()

    def gather(j, kk):
        pltpu.async_copy(x_hbm.at[idxb.at[kk, 0]], rows.at[kk % NBUF],
                         gsems[kk % NBUF])

    def gather_wait(j, kk):
        pltpu.make_async_copy(x_hbm.at[idxb.at[kk, 0]], rows.at[kk % NBUF],
                              gsems[kk % NBUF]).wait()

    def scatter(j, kk):
        pltpu.async_copy(rows.at[kk % NBUF], acc.at[idxb.at[kk, 1]],
                         ssems[kk % NBUF], add=True)

    def scatter_wait(j, kk):
        pltpu.make_async_copy(rows.at[kk % NBUF], acc.at[idxb.at[kk, 1]],
                              ssems[kk % NBUF]).wait()

    def step_one(j, kk, traced):
        def guard(cond, fn):
            if traced:
                pl.when(cond)(fn)
            elif cond:
                fn()

        guard(j + 4 < CPT, lambda: idx_load(j + 4, (kk + 4) % IDXBUF))
        gather_wait(j, kk)
        scatter(j, kk)
        guard((j >= 1) & (j + 2 < CPT),
              lambda: scatter_wait(j - 1, (kk + IDXBUF - 1) % IDXBUF))

        def _advance():
            idx_wait(j + 2, (kk + 2) % IDXBUF)
            gather(j + 2, (kk + 2) % IDXBUF)
        guard(j + 2 < CPT, _advance)

    for t in range(4):
        idx_load(t, t)
    for t in range(2):
        idx_wait(t, t)
        gather(t, t)

    GROUPS = CPT // IDXBUF

    def group(g, carry):
        for kk in range(IDXBUF):
            step_one(g * IDXBUF + kk, kk, True)
        return carry

    lax.fori_loop(0, GROUPS, group, 0)
    for j in range(GROUPS * IDXBUF, CPT):
        step_one(j, j % IDXBUF, False)
    for j in range(CPT - 3, CPT):
        scatter_wait(j, j % IDXBUF)

    plsc.subcore_barrier()

    pltpu.sync_copy(acc.at[pl.ds(row0, ROWS_PER_TILE)],
                    out_hbm.at[pl.ds(cid * N_NODES + row0, ROWS_PER_TILE)])

    @pl.when(sid == NS - 1)
    def _():
        pltpu.sync_copy(acc.at[pl.ds(REM_ROW0, REM_ROWS)],
                        out_hbm.at[pl.ds(cid * N_NODES + REM_ROW0, REM_ROWS)])


@jax.jit
def _sc_aggregate(x, idx, zeros):
    mesh = plsc.VectorSubcoreMesh(core_axis_name="c", subcore_axis_name="s")
    k = pl.kernel(
        _sc_body,
        out_type=jax.ShapeDtypeStruct((NC * N_NODES, D_FEAT), jnp.float32),
        mesh=mesh,
        scratch_types=[
            pltpu.VMEM_SHARED((ACC_ROWS, D_FEAT), jnp.float32),
            pltpu.VMEM((IDXBUF, 2, CHUNK), jnp.int32),
            pltpu.VMEM((NBUF, CHUNK, D_FEAT), jnp.float32),
        ] + [pltpu.SemaphoreType.DMA] * (2 * NBUF + IDXBUF),
    )
    return k(x, idx, zeros)


def _mm_body(p0_ref, p1_ref, w_ref, b_ref, out_ref):
    h = p0_ref[...] + p1_ref[...]
    out = jnp.dot(h, w_ref[...], preferred_element_type=jnp.float32)
    out_ref[...] = jnp.maximum(out + b_ref[...], 0.0)


BLOCK_M = 1000


@jax.jit
def _mm(p0, p1, W, b2d):
    grid = (N_NODES // BLOCK_M,)
    return pl.pallas_call(
        _mm_body,
        grid=grid,
        in_specs=[
            pl.BlockSpec((BLOCK_M, D_FEAT), lambda i: (i, 0)),
            pl.BlockSpec((BLOCK_M, D_FEAT), lambda i: (i, 0)),
            pl.BlockSpec((D_FEAT, D_FEAT), lambda i: (0, 0)),
            pl.BlockSpec((1, D_FEAT), lambda i: (0, 0)),
        ],
        out_specs=pl.BlockSpec((BLOCK_M, D_FEAT), lambda i: (i, 0)),
        out_shape=jax.ShapeDtypeStruct((N_NODES, D_FEAT), jnp.float32),
    )(p0, p1, W, b2d)


def kernel(inputs, edge_index, W, b):
    src = edge_index[0].astype(jnp.int32)
    dst = edge_index[1].astype(jnp.int32)
    per_tile = N_EDGES // NW
    pad_per_tile = EDGES_PER_TILE - per_tile
    pad_dst = jnp.broadcast_to(
        N_NODES + (jnp.arange(pad_per_tile, dtype=jnp.int32) % 16),
        (NW, pad_per_tile))
    src = jnp.concatenate(
        [src.reshape(NW, per_tile),
         jnp.zeros((NW, pad_per_tile), jnp.int32)], axis=1)
    dst = jnp.concatenate([dst.reshape(NW, per_tile), pad_dst], axis=1)
    idx = jnp.stack(
        [src.reshape(NW, CHUNKS_PER_TILE, CHUNK),
         dst.reshape(NW, CHUNKS_PER_TILE, CHUNK)], axis=2)
    idx = idx.reshape(NW * CHUNKS_PER_TILE, 2, CHUNK)
    zeros = jnp.zeros((ROWS_PER_TILE, D_FEAT), jnp.float32)
    parts = _sc_aggregate(inputs, idx, zeros)
    return _mm(parts[:N_NODES], parts[N_NODES:], W, b.reshape(1, D_FEAT))


# speedup vs baseline: 1.8263x; 1.3935x over previous
import jax
import jax.numpy as jnp
from jax import lax
from jax.experimental import pallas as pl
from jax.experimental.pallas import tpu as pltpu
from jax.experimental.pallas import tpu_sc as plsc

N_NODES = 10000
D_FEAT = 128
N_EDGES = 320000

NC = 2
NS = 16
CHUNK = 112
GROUPS = CHUNK // 16

SCAN_ROWS = -(-N_EDGES // CHUNK)
SCAN_ROWS += (-SCAN_ROWS) % NS
E_PAD = SCAN_ROWS * CHUNK
CPT = SCAN_ROWS // NS

HALF = N_NODES // 2
ACC_ROWS = 5016
XROWS = 5008
XBASE = (0, HALF - 8)
TROWS = 320
IDXBUF = 3
NBUF = 3


def _sc_body(x_hbm, idx_hbm, zeros_hbm, out_hbm,
             acc, xsp, idxb, rows, sring, dring, gsem, ssem, *isems):
    cid = lax.axis_index("c")
    sid = lax.axis_index("s")
    lo = cid * HALF

    def refill(slot):
        for v in range(GROUPS):
            sring[slot, pl.ds(16 * v, 16)] = jnp.zeros((16,), jnp.int32)
            dring[slot, pl.ds(16 * v, 16)] = HALF + lax.iota(jnp.int32, 16)

    @pl.when(sid < NS - 1)
    def _():
        pltpu.sync_copy(zeros_hbm, acc.at[pl.ds(sid * TROWS, TROWS)])

    @pl.when(sid == NS - 1)
    def _():
        pltpu.sync_copy(zeros_hbm.at[pl.ds(0, ACC_ROWS - 15 * TROWS)],
                        acc.at[pl.ds(15 * TROWS, ACC_ROWS - 15 * TROWS)])


    base = sid * CPT

    def idx_load(j, kk):
        pltpu.async_copy(idx_hbm.at[base + j], idxb.at[kk], isems[kk])

    def idx_wait(j, kk):
        pltpu.make_async_copy(idx_hbm.at[base + j], idxb.at[kk],
                              isems[kk]).wait()

    def wait_gather():
        pltpu.make_async_copy(xsp.at[sring.at[0]], rows.at[0], gsem).wait()

    def wait_scatter():
        pltpu.make_async_copy(rows.at[0], acc.at[dring.at[0]], ssem).wait()

    def flush(fi):
        b = fi % NBUF
        pltpu.async_copy(xsp.at[sring.at[b]], rows.at[b], gsem)
        wait_gather()
        pltpu.sync_copy(rows.at[b], acc.at[dring.at[b]], add=True)

    def pass_body(p, _):
        xb = p * (HALF - 8)
        slo = p * HALF

        @pl.when(sid < NS - 1)
        def _():
            pltpu.sync_copy(x_hbm.at[pl.ds(xb + sid * TROWS, TROWS)],
                            xsp.at[pl.ds(sid * TROWS, TROWS)])

        @pl.when(sid == NS - 1)
        def _():
            pltpu.sync_copy(
                x_hbm.at[pl.ds(xb + 15 * TROWS, XROWS - 15 * TROWS)],
                xsp.at[pl.ds(15 * TROWS, XROWS - 15 * TROWS)])

        for slot in range(NBUF):
            refill(slot)

        plsc.subcore_barrier()

        for t in range(2):
            idx_load(t, t)

        def scan_chunk(j, kk, off, fi):
            idx_wait(j, kk)

            def _prefetch():
                idx_load(j + 2, (kk + 2) % IDXBUF)
            if isinstance(j, int):
                if j + 2 < CPT:
                    _prefetch()
            else:
                pl.when(j + 2 < CPT)(_prefetch)

            for v in range(GROUPS):
                s = idxb[kk, 0, pl.ds(16 * v, 16)]
                d = idxb[kk, 1, pl.ds(16 * v, 16)]
                m = ((d >= lo) & (d < lo + HALF)
                     & (s >= slo) & (s < slo + HALF))
                sq = s - xb
                dl = d - lo
                c = jnp.sum(jnp.where(m, 1, 0))

                def _do_flush(args):
                    off_, fi_ = args
                    flush(fi_)
                    return 0, fi_ + 1

                off, fi = lax.cond(off + c > CHUNK - 16, _do_flush,
                                   lambda args: args, (off, fi))
                wb = fi % NBUF
                plsc.store_compressed(sring.at[wb, pl.ds(off, 16)], sq,
                                      mask=m)
                plsc.store_compressed(dring.at[wb, pl.ds(off, 16)], dl,
                                      mask=m)
                off = off + c
            refill((fi + 1) % NBUF)
            return off, fi

        def group(g, carry):
            off, fi = carry
            for kk in range(IDXBUF):
                off, fi = scan_chunk(g * IDXBUF + kk, kk, off, fi)
            return off, fi

        ngroups = CPT // IDXBUF
        off, fi = lax.fori_loop(0, ngroups, group, (0, 0))
        for j in range(ngroups * IDXBUF, CPT):
            off, fi = scan_chunk(j, j % IDXBUF, off, fi)

        flush(fi)
        plsc.subcore_barrier()
        return 0

    lax.fori_loop(0, 2, pass_body, 0)

    @pl.when(sid < NS - 1)
    def _():
        pltpu.sync_copy(acc.at[pl.ds(sid * TROWS, TROWS)],
                        out_hbm.at[pl.ds(cid * ACC_ROWS + sid * TROWS, TROWS)])

    @pl.when(sid == NS - 1)
    def _():
        pltpu.sync_copy(
            acc.at[pl.ds(15 * TROWS, ACC_ROWS - 15 * TROWS)],
            out_hbm.at[pl.ds(cid * ACC_ROWS + 15 * TROWS,
                             ACC_ROWS - 15 * TROWS)])


@jax.jit
def _sc_aggregate(x, idx, zeros):
    mesh = plsc.VectorSubcoreMesh(core_axis_name="c", subcore_axis_name="s")
    k = pl.kernel(
        _sc_body,
        out_type=jax.ShapeDtypeStruct((NC * ACC_ROWS, D_FEAT), jnp.float32),
        mesh=mesh,
        compiler_params=pltpu.CompilerParams(needs_layout_passes=False),
        scratch_types=[
            pltpu.VMEM_SHARED((ACC_ROWS, D_FEAT), jnp.float32),
            pltpu.VMEM_SHARED((XROWS, D_FEAT), jnp.float32),
            pltpu.VMEM((IDXBUF, 2, CHUNK), jnp.int32),
            pltpu.VMEM((NBUF, CHUNK, D_FEAT), jnp.float32),
            pltpu.VMEM((NBUF, CHUNK), jnp.int32),
            pltpu.VMEM((NBUF, CHUNK), jnp.int32),
            pltpu.SemaphoreType.DMA,
            pltpu.SemaphoreType.DMA,
        ] + [pltpu.SemaphoreType.DMA] * IDXBUF,
    )
    return k(x, idx, zeros)


def _mm_body(x_ref, agg_ref, w_ref, b_ref, out_ref):
    h = x_ref[...] + agg_ref[...]
    out = jnp.dot(h, w_ref[...], preferred_element_type=jnp.float32)
    out_ref[...] = jnp.maximum(out + b_ref[...], 0.0)


BLOCK_M = 1000


@jax.jit
def _mm(x, agg, W, b2d):
    grid = (N_NODES // BLOCK_M,)
    return pl.pallas_call(
        _mm_body,
        grid=grid,
        in_specs=[
            pl.BlockSpec((BLOCK_M, D_FEAT), lambda i: (i, 0)),
            pl.BlockSpec((BLOCK_M, D_FEAT), lambda i: (i, 0)),
            pl.BlockSpec((D_FEAT, D_FEAT), lambda i: (0, 0)),
            pl.BlockSpec((1, D_FEAT), lambda i: (0, 0)),
        ],
        out_specs=pl.BlockSpec((BLOCK_M, D_FEAT), lambda i: (i, 0)),
        out_shape=jax.ShapeDtypeStruct((N_NODES, D_FEAT), jnp.float32),
    )(x, agg, W, b2d)


def kernel(inputs, edge_index, W, b):
    src = edge_index[0].astype(jnp.int32)
    dst = edge_index[1].astype(jnp.int32)
    pad = E_PAD - N_EDGES
    src = jnp.concatenate([src, jnp.zeros((pad,), jnp.int32)])
    dst = jnp.concatenate([dst, jnp.full((pad,), 2 * N_NODES, jnp.int32)])
    idx = jnp.stack([src.reshape(SCAN_ROWS, CHUNK),
                     dst.reshape(SCAN_ROWS, CHUNK)], axis=1)
    zeros = jnp.zeros((TROWS, D_FEAT), jnp.float32)
    parts = _sc_aggregate(inputs, idx, zeros)
    agg = jnp.concatenate([parts[0:HALF], parts[ACC_ROWS:ACC_ROWS + HALF]])
    return _mm(inputs, agg, W, b.reshape(1, D_FEAT))


# speedup vs baseline: 2.8550x; 1.5633x over previous
import jax
import jax.numpy as jnp
from jax import lax
from jax.experimental import pallas as pl
from jax.experimental.pallas import tpu as pltpu
from jax.experimental.pallas import tpu_sc as plsc

N_NODES = 10000
D_FEAT = 128
N_EDGES = 320000

NC = 2
NS = 16
CHUNK = 112
GROUPS = CHUNK // 16
FL = 128

SCAN_ROWS = -(-N_EDGES // CHUNK)
SCAN_ROWS += (-SCAN_ROWS) % NS
E_PAD = SCAN_ROWS * CHUNK
CPT = SCAN_ROWS // NS

HALF = N_NODES // 2
ACC_ROWS = 5008
XROWS = 5008
XBASE = (0, HALF - 8)
TROWS = 320
IDXBUF = 3
NBUF = 3


def _sc_body(x_hbm, idx_hbm, zeros_hbm, out_hbm,
             acc, xsp, idxb, rows, sring, dring, gsem, ssem, *isems):
    cid = lax.axis_index("c")
    sid = lax.axis_index("s")
    lo = cid * HALF

    zvec = jnp.zeros((16,), jnp.int32)
    dvec = HALF + (lax.iota(jnp.int32, 16) & 7)

    def patch_tail(b, off, nstores):
        for t in range(nstores):
            p0 = jnp.minimum(off + 16 * t, FL - 16)
            sring[b, pl.ds(p0, 16)] = zvec
            dring[b, pl.ds(p0, 16)] = dvec

    @pl.when(sid < NS - 1)
    def _():
        pltpu.sync_copy(zeros_hbm, acc.at[pl.ds(sid * TROWS, TROWS)])

    @pl.when(sid == NS - 1)
    def _():
        pltpu.sync_copy(zeros_hbm.at[pl.ds(0, ACC_ROWS - 15 * TROWS)],
                        acc.at[pl.ds(15 * TROWS, ACC_ROWS - 15 * TROWS)])


    base = sid * CPT

    def idx_load(j, kk):
        pltpu.async_copy(idx_hbm.at[base + j], idxb.at[kk], isems[kk])

    def idx_wait(j, kk):
        pltpu.make_async_copy(idx_hbm.at[base + j], idxb.at[kk],
                              isems[kk]).wait()

    def wait_gather():
        pltpu.make_async_copy(xsp.at[sring.at[0]], rows.at[0], gsem).wait()

    def wait_scatter():
        pltpu.make_async_copy(rows.at[0], acc.at[dring.at[0]], ssem).wait()

    def flush(fi, off, nstores):
        b = fi % NBUF
        bp = (fi + NBUF - 1) % NBUF
        patch_tail(b, off, nstores)

        @pl.when(fi >= 1)
        def _():
            wait_gather()
            pltpu.async_copy(rows.at[bp], acc.at[dring.at[bp]], ssem,
                             add=True)

        @pl.when(fi >= 2)
        def _():
            wait_scatter()

        pltpu.async_copy(xsp.at[sring.at[b]], rows.at[b], gsem)

    def pass_body(p, _):
        xb = p * (HALF - 8)
        slo = p * HALF

        @pl.when(sid < NS - 1)
        def _():
            pltpu.sync_copy(x_hbm.at[pl.ds(xb + sid * TROWS, TROWS)],
                            xsp.at[pl.ds(sid * TROWS, TROWS)])

        @pl.when(sid == NS - 1)
        def _():
            pltpu.sync_copy(
                x_hbm.at[pl.ds(xb + 15 * TROWS, XROWS - 15 * TROWS)],
                xsp.at[pl.ds(15 * TROWS, XROWS - 15 * TROWS)])

        plsc.subcore_barrier()

        for t in range(2):
            idx_load(t, t)

        def scan_chunk(j, kk, off, fi):
            idx_wait(j, kk)

            def _prefetch():
                idx_load(j + 2, (kk + 2) % IDXBUF)
            if isinstance(j, int):
                if j + 2 < CPT:
                    _prefetch()
            else:
                pl.when(j + 2 < CPT)(_prefetch)

            for v in range(GROUPS):
                s = idxb[kk, 0, pl.ds(16 * v, 16)]
                d = idxb[kk, 1, pl.ds(16 * v, 16)]
                m = ((d >= lo) & (d < lo + HALF)
                     & (s >= slo) & (s < slo + HALF))
                sq = s - xb
                dl = d - lo
                c = jnp.sum(jnp.where(m, 1, 0))

                def _do_flush(args):
                    off_, fi_ = args
                    flush(fi_, off_, 2)
                    return 0, fi_ + 1

                off, fi = lax.cond(off + c > FL - 16, _do_flush,
                                   lambda args: args, (off, fi))
                wb = fi % NBUF
                plsc.store_compressed(sring.at[wb, pl.ds(off, 16)], sq,
                                      mask=m)
                plsc.store_compressed(dring.at[wb, pl.ds(off, 16)], dl,
                                      mask=m)
                off = off + c
            return off, fi

        def group(g, carry):
            off, fi = carry
            for kk in range(IDXBUF):
                off, fi = scan_chunk(g * IDXBUF + kk, kk, off, fi)
            return off, fi

        ngroups = CPT // IDXBUF
        off, fi = lax.fori_loop(0, ngroups, group, (0, 0))
        for j in range(ngroups * IDXBUF, CPT):
            off, fi = scan_chunk(j, j % IDXBUF, off, fi)

        flush(fi, off, 8)
        wait_gather()
        pltpu.async_copy(rows.at[fi % NBUF], acc.at[dring.at[fi % NBUF]],
                         ssem, add=True)

        @pl.when(fi >= 1)
        def _():
            wait_scatter()

        wait_scatter()
        plsc.subcore_barrier()
        return 0

    lax.fori_loop(0, 2, pass_body, 0)

    @pl.when(sid < NS - 1)
    def _():
        pltpu.sync_copy(acc.at[pl.ds(sid * TROWS, TROWS)],
                        out_hbm.at[pl.ds(cid * ACC_ROWS + sid * TROWS, TROWS)])

    @pl.when(sid == NS - 1)
    def _():
        pltpu.sync_copy(
            acc.at[pl.ds(15 * TROWS, ACC_ROWS - 15 * TROWS)],
            out_hbm.at[pl.ds(cid * ACC_ROWS + 15 * TROWS,
                             ACC_ROWS - 15 * TROWS)])


@jax.jit
def _sc_aggregate(x, idx, zeros):
    mesh = plsc.VectorSubcoreMesh(core_axis_name="c", subcore_axis_name="s")
    k = pl.kernel(
        _sc_body,
        out_type=jax.ShapeDtypeStruct((NC * ACC_ROWS, D_FEAT), jnp.float32),
        mesh=mesh,
        compiler_params=pltpu.CompilerParams(needs_layout_passes=False),
        scratch_types=[
            pltpu.VMEM_SHARED((ACC_ROWS, D_FEAT), jnp.float32),
            pltpu.VMEM_SHARED((XROWS, D_FEAT), jnp.float32),
            pltpu.VMEM((IDXBUF, 2, CHUNK), jnp.int32),
            pltpu.VMEM((NBUF, FL, D_FEAT), jnp.float32),
            pltpu.VMEM((NBUF, FL), jnp.int32),
            pltpu.VMEM((NBUF, FL), jnp.int32),
            pltpu.SemaphoreType.DMA,
            pltpu.SemaphoreType.DMA,
        ] + [pltpu.SemaphoreType.DMA] * IDXBUF,
    )
    return k(x, idx, zeros)


def _mm_body(x_ref, agg_ref, w_ref, b_ref, out_ref):
    h = x_ref[...] + agg_ref[...]
    out = jnp.dot(h, w_ref[...], preferred_element_type=jnp.float32)
    out_ref[...] = jnp.maximum(out + b_ref[...], 0.0)


BLOCK_M = 1000


@jax.jit
def _mm(x, agg, W, b2d):
    grid = (N_NODES // BLOCK_M,)
    return pl.pallas_call(
        _mm_body,
        grid=grid,
        in_specs=[
            pl.BlockSpec((BLOCK_M, D_FEAT), lambda i: (i, 0)),
            pl.BlockSpec((BLOCK_M, D_FEAT), lambda i: (i, 0)),
            pl.BlockSpec((D_FEAT, D_FEAT), lambda i: (0, 0)),
            pl.BlockSpec((1, D_FEAT), lambda i: (0, 0)),
        ],
        out_specs=pl.BlockSpec((BLOCK_M, D_FEAT), lambda i: (i, 0)),
        out_shape=jax.ShapeDtypeStruct((N_NODES, D_FEAT), jnp.float32),
    )(x, agg, W, b2d)


def kernel(inputs, edge_index, W, b):
    src = edge_index[0].astype(jnp.int32)
    dst = edge_index[1].astype(jnp.int32)
    pad = E_PAD - N_EDGES
    src = jnp.concatenate([src, jnp.zeros((pad,), jnp.int32)])
    dst = jnp.concatenate([dst, jnp.full((pad,), 2 * N_NODES, jnp.int32)])
    idx = jnp.stack([src.reshape(SCAN_ROWS, CHUNK),
                     dst.reshape(SCAN_ROWS, CHUNK)], axis=1)
    zeros = jnp.zeros((TROWS, D_FEAT), jnp.float32)
    parts = _sc_aggregate(inputs, idx, zeros)
    agg = jnp.concatenate([parts[0:HALF], parts[ACC_ROWS:ACC_ROWS + HALF]])
    return _mm(inputs, agg, W, b.reshape(1, D_FEAT))


# speedup vs baseline: 3.0289x; 1.0609x over previous
import jax
import jax.numpy as jnp
from jax import lax
from jax.experimental import pallas as pl
from jax.experimental.pallas import tpu as pltpu
from jax.experimental.pallas import tpu_sc as plsc

N_NODES = 10000
D_FEAT = 128
N_EDGES = 320000

NC = 2
NS = 16
CHUNK = 112
GROUPS = CHUNK // 16
FL = 128

SCAN_ROWS = -(-N_EDGES // CHUNK)
SCAN_ROWS += (-SCAN_ROWS) % NS
E_PAD = SCAN_ROWS * CHUNK
CPT = SCAN_ROWS // NS

HALF = N_NODES // 2
ACC_ROWS = 5008
XROWS = 5008
XBASE = (0, HALF - 8)
TROWS = 320
IDXBUF = 3
NBUF = 3


def _sc_body(x_hbm, src_hbm, dst_hbm, zeros_hbm, out_hbm,
             acc, xsp, idxb, rows, sring, dring, gsem, ssem, *isems):
    cid = lax.axis_index("c")
    sid = lax.axis_index("s")
    lo = cid * HALF

    zvec = jnp.zeros((16,), jnp.int32)
    dvec = HALF + (lax.iota(jnp.int32, 16) & 7)

    def patch_tail(b, off, nstores):
        for t in range(nstores):
            p0 = jnp.minimum(off + 16 * t, FL - 16)
            sring[b, pl.ds(p0, 16)] = zvec
            dring[b, pl.ds(p0, 16)] = dvec

    @pl.when(sid < NS - 1)
    def _():
        pltpu.sync_copy(zeros_hbm, acc.at[pl.ds(sid * TROWS, TROWS)])

    @pl.when(sid == NS - 1)
    def _():
        pltpu.sync_copy(zeros_hbm.at[pl.ds(0, ACC_ROWS - 15 * TROWS)],
                        acc.at[pl.ds(15 * TROWS, ACC_ROWS - 15 * TROWS)])


    base = sid * CPT

    def idx_load(j, kk):
        e0 = (base + j) * CHUNK
        pltpu.async_copy(src_hbm.at[pl.ds(e0, CHUNK)], idxb.at[kk, 0],
                         isems[kk])
        pltpu.async_copy(dst_hbm.at[pl.ds(e0, CHUNK)], idxb.at[kk, 1],
                         isems[kk])

    def idx_wait(j, kk):
        e0 = (base + j) * CHUNK
        pltpu.make_async_copy(src_hbm.at[pl.ds(e0, CHUNK)], idxb.at[kk, 0],
                              isems[kk]).wait()
        pltpu.make_async_copy(dst_hbm.at[pl.ds(e0, CHUNK)], idxb.at[kk, 1],
                              isems[kk]).wait()

    def wait_gather():
        pltpu.make_async_copy(xsp.at[sring.at[0]], rows.at[0], gsem).wait()

    def wait_scatter():
        pltpu.make_async_copy(rows.at[0], acc.at[dring.at[0]], ssem).wait()

    def flush(fi, off, nstores):
        b = fi % NBUF
        bp = (fi + NBUF - 1) % NBUF
        patch_tail(b, off, nstores)

        @pl.when(fi >= 1)
        def _():
            wait_gather()
            pltpu.async_copy(rows.at[bp], acc.at[dring.at[bp]], ssem,
                             add=True)

        @pl.when(fi >= 2)
        def _():
            wait_scatter()

        pltpu.async_copy(xsp.at[sring.at[b]], rows.at[b], gsem)

    def pass_body(p, _):
        xb = p * (HALF - 8)
        slo = p * HALF

        @pl.when(sid < NS - 1)
        def _():
            pltpu.sync_copy(x_hbm.at[pl.ds(xb + sid * TROWS, TROWS)],
                            xsp.at[pl.ds(sid * TROWS, TROWS)])

        @pl.when(sid == NS - 1)
        def _():
            pltpu.sync_copy(
                x_hbm.at[pl.ds(xb + 15 * TROWS, XROWS - 15 * TROWS)],
                xsp.at[pl.ds(15 * TROWS, XROWS - 15 * TROWS)])

        plsc.subcore_barrier()

        for t in range(2):
            idx_load(t, t)

        def scan_chunk(j, kk, off, fi):
            idx_wait(j, kk)

            def _prefetch():
                idx_load(j + 2, (kk + 2) % IDXBUF)
            if isinstance(j, int):
                if j + 2 < CPT:
                    _prefetch()
            else:
                pl.when(j + 2 < CPT)(_prefetch)

            for v in range(GROUPS):
                s = idxb[kk, 0, pl.ds(16 * v, 16)]
                d = idxb[kk, 1, pl.ds(16 * v, 16)]
                m = ((d >= lo) & (d < lo + HALF)
                     & (s >= slo) & (s < slo + HALF))
                sq = s - xb
                dl = d - lo
                c = jnp.sum(jnp.where(m, 1, 0))

                def _do_flush(args):
                    off_, fi_ = args
                    flush(fi_, off_, 2)
                    return 0, fi_ + 1

                off, fi = lax.cond(off + c > FL - 16, _do_flush,
                                   lambda args: args, (off, fi))
                wb = fi % NBUF
                plsc.store_compressed(sring.at[wb, pl.ds(off, 16)], sq,
                                      mask=m)
                plsc.store_compressed(dring.at[wb, pl.ds(off, 16)], dl,
                                      mask=m)
                off = off + c
            return off, fi

        def group(g, carry):
            off, fi = carry
            for kk in range(IDXBUF):
                off, fi = scan_chunk(g * IDXBUF + kk, kk, off, fi)
            return off, fi

        ngroups = CPT // IDXBUF
        off, fi = lax.fori_loop(0, ngroups, group, (0, 0))
        for j in range(ngroups * IDXBUF, CPT):
            off, fi = scan_chunk(j, j % IDXBUF, off, fi)

        flush(fi, off, 8)
        wait_gather()
        pltpu.async_copy(rows.at[fi % NBUF], acc.at[dring.at[fi % NBUF]],
                         ssem, add=True)

        @pl.when(fi >= 1)
        def _():
            wait_scatter()

        wait_scatter()
        plsc.subcore_barrier()
        return 0

    lax.fori_loop(0, 2, pass_body, 0)

    @pl.when(sid < NS - 1)
    def _():
        pltpu.sync_copy(acc.at[pl.ds(sid * TROWS, TROWS)],
                        out_hbm.at[pl.ds(cid * HALF + sid * TROWS, TROWS)])

    @pl.when(sid == NS - 1)
    def _():
        pltpu.sync_copy(
            acc.at[pl.ds(15 * TROWS, HALF - 15 * TROWS)],
            out_hbm.at[pl.ds(cid * HALF + 15 * TROWS, HALF - 15 * TROWS)])


@jax.jit
def _sc_aggregate(x, src, dst, zeros):
    mesh = plsc.VectorSubcoreMesh(core_axis_name="c", subcore_axis_name="s")
    k = pl.kernel(
        _sc_body,
        out_type=jax.ShapeDtypeStruct((N_NODES, D_FEAT), jnp.float32),
        mesh=mesh,
        compiler_params=pltpu.CompilerParams(needs_layout_passes=False),
        scratch_types=[
            pltpu.VMEM_SHARED((ACC_ROWS, D_FEAT), jnp.float32),
            pltpu.VMEM_SHARED((XROWS, D_FEAT), jnp.float32),
            pltpu.VMEM((IDXBUF, 2, CHUNK), jnp.int32),
            pltpu.VMEM((NBUF, FL, D_FEAT), jnp.float32),
            pltpu.VMEM((NBUF, FL), jnp.int32),
            pltpu.VMEM((NBUF, FL), jnp.int32),
            pltpu.SemaphoreType.DMA,
            pltpu.SemaphoreType.DMA,
        ] + [pltpu.SemaphoreType.DMA] * IDXBUF,
    )
    return k(x, src, dst, zeros)


def _mm_body(x_ref, agg_ref, w_ref, b_ref, out_ref):
    h = x_ref[...] + agg_ref[...]
    out = jnp.dot(h, w_ref[...], preferred_element_type=jnp.float32)
    out_ref[...] = jnp.maximum(out + b_ref[...], 0.0)


BLOCK_M = 1000


@jax.jit
def _mm(x, agg, W, b2d):
    grid = (N_NODES // BLOCK_M,)
    return pl.pallas_call(
        _mm_body,
        grid=grid,
        in_specs=[
            pl.BlockSpec((BLOCK_M, D_FEAT), lambda i: (i, 0)),
            pl.BlockSpec((BLOCK_M, D_FEAT), lambda i: (i, 0)),
            pl.BlockSpec((D_FEAT, D_FEAT), lambda i: (0, 0)),
            pl.BlockSpec((1, D_FEAT), lambda i: (0, 0)),
        ],
        out_specs=pl.BlockSpec((BLOCK_M, D_FEAT), lambda i: (i, 0)),
        out_shape=jax.ShapeDtypeStruct((N_NODES, D_FEAT), jnp.float32),
    )(x, agg, W, b2d)


def kernel(inputs, edge_index, W, b):
    src = edge_index[0].astype(jnp.int32)
    dst = edge_index[1].astype(jnp.int32)
    pad = E_PAD - N_EDGES
    src = jnp.concatenate([src, jnp.zeros((pad,), jnp.int32)])
    dst = jnp.concatenate([dst, jnp.full((pad,), 2 * N_NODES, jnp.int32)])
    zeros = jnp.zeros((TROWS, D_FEAT), jnp.float32)
    agg = _sc_aggregate(inputs, src, dst, zeros)
    return _mm(inputs, agg, W, b.reshape(1, D_FEAT))
